# Initial kernel scaffold; baseline (speedup 1.0000x reference)
import jax
import jax.numpy as jnp
from jax.experimental import pallas as pl


def kernel(x, edge_index, mask):
    raise NotImplementedError("write your pallas kernel here")


# speedup vs baseline: 5.2782x; 5.2782x over previous
import jax
import jax.numpy as jnp
from jax import lax
from jax.experimental import pallas as pl
from jax.experimental.pallas import tpu as pltpu
from jax.experimental.pallas import tpu_sc as plsc

N_NODES = 10000
N_EDGES = 160000
D_FEAT = 256
N_ITER = 40

NCORE = 2
NSUB = 16
HALF = D_FEAT // NCORE
EPT = N_EDGES // NSUB
ECH = 125
NCHUNK = EPT // ECH
NPAD = 10240
NPT = NPAD // NSUB
NROW = 32
NNCH = NPT // NROW
LANES = 16


def _zero_nbuf(nbuf):
    def _zrow(n, _):
        def _zcol(j, _):
            nbuf[n, pl.ds(j * LANES, LANES)] = jnp.zeros((LANES,), jnp.float32)
            return None
        return lax.fori_loop(0, HALF // LANES, _zcol, None)
    lax.fori_loop(0, NROW, _zrow, None)


def _fp_kernel(xm, cco, bco, gcol4, row4, out, ytab, acc,
               colv, rowv, ebuf, nbuf, cbuf, sem):
    c = lax.axis_index("c")
    s = lax.axis_index("s")
    base = s * NPT
    ybase = c * (2 * NPAD)
    ndmy = ybase + base
    ncst = ybase + NPAD + base

    pltpu.sync_copy(gcol4.at[c, s], colv)
    pltpu.sync_copy(row4.at[s], rowv)

    _zero_nbuf(nbuf)
    def _init(k, _):
        n0 = k * NROW
        pltpu.sync_copy(nbuf, ytab.at[pl.ds(ndmy + n0, NROW)])
        pltpu.sync_copy(xm.at[c, pl.ds(base + n0, NROW)], cbuf)
        pltpu.sync_copy(cbuf, ytab.at[pl.ds(ncst + n0, NROW)])
        return None
    lax.fori_loop(0, NNCH, _init, None)
    plsc.subcore_barrier()

    def edge_phase():
        _zero_nbuf(nbuf)
        def _clr(k, _):
            pltpu.sync_copy(nbuf, acc.at[pl.ds(base + k * NROW, NROW)])
            return None
        lax.fori_loop(0, NNCH, _clr, None)
        plsc.subcore_barrier()

        def _echunk(j, _):
            pltpu.async_copy(ytab.at[colv.at[j]], ebuf, sem).wait()
            pltpu.sync_copy(ebuf, acc.at[rowv.at[j]], add=True)
            return None
        lax.fori_loop(0, NCHUNK, _echunk, None)
        plsc.subcore_barrier()

    def node_phase(coef_hbm, dst_is_out):
        def _nchunk(k, _):
            n0 = base + k * NROW
            pltpu.sync_copy(acc.at[pl.ds(n0, NROW)], nbuf)
            pltpu.sync_copy(coef_hbm.at[c, pl.ds(n0, NROW)], cbuf)

            def _frow(n, _):
                def _fcol(j, _):
                    sl = pl.ds(j * LANES, LANES)
                    nbuf[n, sl] = cbuf[n, sl] * nbuf[n, sl]
                    return None
                return lax.fori_loop(0, HALF // LANES, _fcol, None)
            lax.fori_loop(0, NROW, _frow, None)

            if dst_is_out:
                pltpu.sync_copy(nbuf, out.at[c, pl.ds(n0, NROW)])
            else:
                pltpu.sync_copy(nbuf, ytab.at[pl.ds(ybase + n0, NROW)])
            return None
        lax.fori_loop(0, NNCH, _nchunk, None)
        plsc.subcore_barrier()

    def _iter(t, _):
        edge_phase()
        node_phase(cco, False)
        return None
    lax.fori_loop(0, N_ITER - 1, _iter, None)
    edge_phase()
    node_phase(bco, True)


def kernel(x, edge_index, mask):
    x = x.astype(jnp.float32)
    row = edge_index[0].astype(jnp.int32)
    col = edge_index[1].astype(jnp.int32)

    deg = jnp.zeros((N_NODES,), jnp.float32).at[row].add(1.0)
    dis = jnp.where(deg > 0, lax.rsqrt(jnp.maximum(deg, 1.0)), 0.0)

    maskf = mask.astype(jnp.float32)
    xm_full = x * (maskf * dis)[:, None]
    ones = jnp.ones((1, D_FEAT), jnp.float32)
    cco_full = ((1.0 - maskf) * dis * dis)[:, None] * ones
    bco_full = ((1.0 - maskf) * dis)[:, None] * ones

    def halves(a):
        a = a.reshape(N_NODES, NCORE, HALF).transpose(1, 0, 2)
        return jnp.pad(a, ((0, 0), (0, NPAD - N_NODES), (0, 0)))

    xm = halves(xm_full)
    cco = halves(cco_full)
    bco = halves(bco_full)

    gcol = col + mask[col].astype(jnp.int32) * NPAD
    gcol_t = gcol.reshape(NSUB, NCHUNK, ECH)
    gcol4 = jnp.stack([gcol_t, gcol_t + 2 * NPAD], axis=0)
    row4 = row.reshape(NSUB, NCHUNK, ECH)

    f32 = jnp.float32
    run = pl.kernel(
        _fp_kernel,
        out_type=(
            jax.ShapeDtypeStruct((NCORE, NPAD, HALF), f32),
            jax.ShapeDtypeStruct((2 * NCORE * NPAD, HALF), f32),
        ),
        mesh=plsc.VectorSubcoreMesh(core_axis_name="c", subcore_axis_name="s"),
        scratch_types=[
            pltpu.VMEM_SHARED((NPAD, HALF), f32),
            pltpu.VMEM((NCHUNK, ECH), jnp.int32),
            pltpu.VMEM((NCHUNK, ECH), jnp.int32),
            pltpu.VMEM((ECH, HALF), f32),
            pltpu.VMEM((NROW, HALF), f32),
            pltpu.VMEM((NROW, HALF), f32),
            pltpu.SemaphoreType.DMA,
        ],
    )
    out_halves, _ = run(xm, cco, bco, gcol4, row4)
    out = out_halves.transpose(1, 0, 2).reshape(NPAD, D_FEAT)[:N_NODES]
    return jnp.where(mask[:, None], x, out)


# speedup vs baseline: 5.6303x; 1.0667x over previous
import jax
import jax.numpy as jnp
from jax import lax
from jax.experimental import pallas as pl
from jax.experimental.pallas import tpu as pltpu
from jax.experimental.pallas import tpu_sc as plsc

N_NODES = 10000
N_EDGES = 160000
D_FEAT = 256
N_ITER = 40

NCORE = 2
NSUB = 16
HALF = D_FEAT // NCORE
EPT = N_EDGES // NSUB
ECH = 125
NCHUNK = EPT // ECH
GRP = 8
NGRP = NCHUNK // GRP
NPAD = 10240
NPT = NPAD // NSUB
NROW = 32
NNCH = NPT // NROW
LANES = 16


def _fp_kernel(xm, cco, bco, gcol5, row5, out, ytab, acc,
               cidx, ridx, ebuf0, ebuf1, nbuf, cbuf, zbuf,
               sg0, sg1, ss0, ss1):
    c = lax.axis_index("c")
    s = lax.axis_index("s")
    base = s * NPT
    ybase = c * (2 * NPAD)
    ndmy = ybase + base
    ncst = ybase + NPAD + base

    def _zrow(n, _):
        def _zcol(j, _):
            zbuf[n, pl.ds(j * LANES, LANES)] = jnp.zeros((LANES,), jnp.float32)
            return None
        return lax.fori_loop(0, HALF // LANES, _zcol, None)
    lax.fori_loop(0, NROW, _zrow, None)

    def _init(k, _):
        n0 = k * NROW
        pltpu.sync_copy(zbuf, ytab.at[pl.ds(ndmy + n0, NROW)])
        pltpu.sync_copy(zbuf, acc.at[pl.ds(base + n0, NROW)])
        pltpu.sync_copy(xm.at[c, pl.ds(base + n0, NROW)], cbuf)
        pltpu.sync_copy(cbuf, ytab.at[pl.ds(ncst + n0, NROW)])
        return None
    lax.fori_loop(0, NNCH, _init, None)
    plsc.subcore_barrier()

    def edge_phase():
        def _egroup(g, _):
            pltpu.sync_copy(gcol5.at[c, s, g], cidx)
            pltpu.sync_copy(row5.at[s, g], ridx)

            def _epair(p, _):
                j0 = 2 * p
                g0 = pltpu.async_copy(ytab.at[cidx.at[j0]], ebuf0, sg0)
                g1 = pltpu.async_copy(ytab.at[cidx.at[j0 + 1]], ebuf1, sg1)
                g0.wait()
                g1.wait()
                s0 = pltpu.async_copy(ebuf0, acc.at[ridx.at[j0]], ss0,
                                      add=True)
                s1 = pltpu.async_copy(ebuf1, acc.at[ridx.at[j0 + 1]], ss1,
                                      add=True)
                s0.wait()
                s1.wait()
                return None
            lax.fori_loop(0, GRP // 2, _epair, None)
            return None
        lax.fori_loop(0, NGRP, _egroup, None)
        plsc.subcore_barrier()

    def node_phase(coef_hbm, dst_is_out):
        def _nchunk(k, _):
            n0 = base + k * NROW
            pltpu.sync_copy(acc.at[pl.ds(n0, NROW)], nbuf)
            pltpu.sync_copy(zbuf, acc.at[pl.ds(n0, NROW)])
            pltpu.sync_copy(coef_hbm.at[c, pl.ds(n0, NROW)], cbuf)

            def _frow(n, _):
                def _fcol(j, _):
                    sl = pl.ds(j * LANES, LANES)
                    nbuf[n, sl] = cbuf[n, sl] * nbuf[n, sl]
                    return None
                return lax.fori_loop(0, HALF // LANES, _fcol, None)
            lax.fori_loop(0, NROW, _frow, None)

            if dst_is_out:
                pltpu.sync_copy(nbuf, out.at[c, pl.ds(n0, NROW)])
            else:
                pltpu.sync_copy(nbuf, ytab.at[pl.ds(ybase + n0, NROW)])
            return None
        lax.fori_loop(0, NNCH, _nchunk, None)
        plsc.subcore_barrier()

    def _iter(t, _):
        edge_phase()
        node_phase(cco, False)
        return None
    lax.fori_loop(0, N_ITER - 1, _iter, None)
    edge_phase()
    node_phase(bco, True)


def kernel(x, edge_index, mask):
    x = x.astype(jnp.float32)
    row = edge_index[0].astype(jnp.int32)
    col = edge_index[1].astype(jnp.int32)

    deg = jnp.zeros((N_NODES,), jnp.float32).at[row].add(1.0)
    dis = jnp.where(deg > 0, lax.rsqrt(jnp.maximum(deg, 1.0)), 0.0)

    maskf = mask.astype(jnp.float32)
    xm_full = x * (maskf * dis)[:, None]
    ones = jnp.ones((1, D_FEAT), jnp.float32)
    cco_full = ((1.0 - maskf) * dis * dis)[:, None] * ones
    bco_full = ((1.0 - maskf) * dis)[:, None] * ones

    def halves(a):
        a = a.reshape(N_NODES, NCORE, HALF).transpose(1, 0, 2)
        return jnp.pad(a, ((0, 0), (0, NPAD - N_NODES), (0, 0)))

    xm = halves(xm_full)
    cco = halves(cco_full)
    bco = halves(bco_full)

    gcol = col + mask[col].astype(jnp.int32) * NPAD
    gcol_t = gcol.reshape(NSUB, NGRP, GRP, ECH)
    gcol5 = jnp.stack([gcol_t, gcol_t + 2 * NPAD], axis=0)
    row5 = row.reshape(NSUB, NGRP, GRP, ECH)

    f32 = jnp.float32
    run = pl.kernel(
        _fp_kernel,
        out_type=(
            jax.ShapeDtypeStruct((NCORE, NPAD, HALF), f32),
            jax.ShapeDtypeStruct((2 * NCORE * NPAD, HALF), f32),
        ),
        mesh=plsc.VectorSubcoreMesh(core_axis_name="c", subcore_axis_name="s"),
        scratch_types=[
            pltpu.VMEM_SHARED((NPAD, HALF), f32),
            pltpu.VMEM((GRP, ECH), jnp.int32),
            pltpu.VMEM((GRP, ECH), jnp.int32),
            pltpu.VMEM((ECH, HALF), f32),
            pltpu.VMEM((ECH, HALF), f32),
            pltpu.VMEM((NROW, HALF), f32),
            pltpu.VMEM((NROW, HALF), f32),
            pltpu.VMEM((NROW, HALF), f32),
            pltpu.SemaphoreType.DMA,
            pltpu.SemaphoreType.DMA,
            pltpu.SemaphoreType.DMA,
            pltpu.SemaphoreType.DMA,
        ],
    )
    out_halves, _ = run(xm, cco, bco, gcol5, row5)
    out = out_halves.transpose(1, 0, 2).reshape(NPAD, D_FEAT)[:N_NODES]
    return jnp.where(mask[:, None], x, out)


# speedup vs baseline: 7.0386x; 1.2501x over previous
import jax
import jax.numpy as jnp
from jax import lax
from jax.experimental import pallas as pl
from jax.experimental.pallas import tpu as pltpu
from jax.experimental.pallas import tpu_sc as plsc

N_NODES = 10000
N_EDGES = 160000
D_FEAT = 256
N_ITER = 40

NCORE = 2
NSUB = 16
HALF = D_FEAT // NCORE
EPT = N_EDGES // NSUB
ECH = 125
NCHUNK = EPT // ECH
GRP = 8
NGRP = NCHUNK // GRP
NPAD = 10240
NPT = NPAD // NSUB
NROW = 32
NNCH = NPT // NROW
LANES = 16


def _fp_kernel(xm, ccov, bcov, gcol5, row5, out, ytab, acc,
               cidx, ridx, ebuf0, ebuf1, nbuf0, nbuf1, zbuf, cvec,
               sg0, sg1, ss0, ss1, sn0, sn1, sz):
    c = lax.axis_index("c")
    s = lax.axis_index("s")
    base = s * NPT
    ybase = c * (2 * NPAD)
    ndmy = ybase + base
    ncst = ybase + NPAD + base
    ebufs = (ebuf0, ebuf1)
    nbufs = (nbuf0, nbuf1)
    sgs = (sg0, sg1)
    sss = (ss0, ss1)
    sns = (sn0, sn1)

    def _zrow(n, _):
        def _zcol(j, _):
            zbuf[n, pl.ds(j * LANES, LANES)] = jnp.zeros((LANES,), jnp.float32)
            return None
        return lax.fori_loop(0, HALF // LANES, _zcol, None)
    lax.fori_loop(0, NROW, _zrow, None)

    pltpu.sync_copy(ccov.at[c, 0, pl.ds(base, NPT)], cvec)

    def _init(k, _):
        n0 = k * NROW
        pltpu.sync_copy(zbuf, ytab.at[pl.ds(ndmy + n0, NROW)])
        pltpu.sync_copy(zbuf, acc.at[pl.ds(base + n0, NROW)])
        pltpu.sync_copy(xm.at[c, pl.ds(base + n0, NROW)], nbuf0)
        pltpu.sync_copy(nbuf0, ytab.at[pl.ds(ncst + n0, NROW)])
        return None
    lax.fori_loop(0, NNCH, _init, None)
    plsc.subcore_barrier()

    def edge_phase():
        def _egroup(g, _):
            pltpu.sync_copy(gcol5.at[c, s, g], cidx)
            pltpu.sync_copy(row5.at[s, g], ridx)

            gh = [None, None]
            sh = [None, None]
            gh[0] = pltpu.async_copy(ytab.at[cidx.at[0]], ebuf0, sg0)
            for j in range(GRP):
                b = j % 2
                gh[b].wait()
                sh[b] = pltpu.async_copy(ebufs[b], acc.at[ridx.at[j]],
                                         sss[b], add=True)
                if j + 1 < GRP:
                    if sh[1 - b] is not None:
                        sh[1 - b].wait()
                    gh[1 - b] = pltpu.async_copy(ytab.at[cidx.at[j + 1]],
                                                 ebufs[1 - b], sgs[1 - b])
            sh[0].wait()
            sh[1].wait()
            return None
        lax.fori_loop(0, NGRP, _egroup, None)
        plsc.subcore_barrier()

    def node_phase(dst_is_out):
        rd = [None, None]
        wr = [None, None]
        rd[0] = pltpu.async_copy(acc.at[pl.ds(base, NROW)], nbuf0, sn0)
        for k in range(NNCH):
            b = k % 2
            n0 = base + k * NROW
            rd[b].wait()
            zr = pltpu.async_copy(zbuf, acc.at[pl.ds(n0, NROW)], sz)
            if k + 1 < NNCH:
                if wr[1 - b] is not None:
                    wr[1 - b].wait()
                rd[1 - b] = pltpu.async_copy(
                    acc.at[pl.ds(base + (k + 1) * NROW, NROW)],
                    nbufs[1 - b], sns[1 - b])

            nb = nbufs[b]
            def _frow(nn, _):
                cv16 = cvec[pl.ds(k * NROW + nn * LANES, LANES)]
                for l in range(LANES):
                    cn = cv16[l]
                    n = nn * LANES + l
                    def _fcol(j, _):
                        sl = pl.ds(j * LANES, LANES)
                        nb[n, sl] = nb[n, sl] * cn
                        return None
                    lax.fori_loop(0, HALF // LANES, _fcol, None)
                return None
            lax.fori_loop(0, NROW // LANES, _frow, None)
            zr.wait()

            if dst_is_out:
                wr[b] = pltpu.async_copy(nb, out.at[c, pl.ds(n0, NROW)],
                                         sns[b])
            else:
                wr[b] = pltpu.async_copy(
                    nb, ytab.at[pl.ds(ybase + n0, NROW)], sns[b])
        if wr[0] is not None:
            wr[0].wait()
        if wr[1] is not None:
            wr[1].wait()
        plsc.subcore_barrier()

    def _iter(t, _):
        edge_phase()
        node_phase(False)
        return None
    lax.fori_loop(0, N_ITER - 1, _iter, None)
    pltpu.sync_copy(bcov.at[c, 0, pl.ds(base, NPT)], cvec)
    edge_phase()
    node_phase(True)


def kernel(x, edge_index, mask):
    x = x.astype(jnp.float32)
    row = edge_index[0].astype(jnp.int32)
    col = edge_index[1].astype(jnp.int32)

    deg = jnp.zeros((N_NODES,), jnp.float32).at[row].add(1.0)
    dis = jnp.where(deg > 0, lax.rsqrt(jnp.maximum(deg, 1.0)), 0.0)

    maskf = mask.astype(jnp.float32)
    xm_full = x * (maskf * dis)[:, None]
    cco = (1.0 - maskf) * dis * dis
    bco = (1.0 - maskf) * dis

    def halves(a):
        a = a.reshape(N_NODES, NCORE, HALF).transpose(1, 0, 2)
        return jnp.pad(a, ((0, 0), (0, NPAD - N_NODES), (0, 0)))

    xm = halves(xm_full)

    def coefv(a):
        a = jnp.pad(a, (0, NPAD - N_NODES))
        return jnp.broadcast_to(a[None, None, :], (NCORE, 1, NPAD))

    ccov = coefv(cco)
    bcov = coefv(bco)

    gcol = col + mask[col].astype(jnp.int32) * NPAD
    gcol_t = gcol.reshape(NSUB, NGRP, GRP, ECH)
    gcol5 = jnp.stack([gcol_t, gcol_t + 2 * NPAD], axis=0)
    row5 = row.reshape(NSUB, NGRP, GRP, ECH)

    f32 = jnp.float32
    run = pl.kernel(
        _fp_kernel,
        out_type=(
            jax.ShapeDtypeStruct((NCORE, NPAD, HALF), f32),
            jax.ShapeDtypeStruct((2 * NCORE * NPAD, HALF), f32),
        ),
        mesh=plsc.VectorSubcoreMesh(core_axis_name="c", subcore_axis_name="s"),
        scratch_types=[
            pltpu.VMEM_SHARED((NPAD, HALF), f32),
            pltpu.VMEM((GRP, ECH), jnp.int32),
            pltpu.VMEM((GRP, ECH), jnp.int32),
            pltpu.VMEM((ECH, HALF), f32),
            pltpu.VMEM((ECH, HALF), f32),
            pltpu.VMEM((NROW, HALF), f32),
            pltpu.VMEM((NROW, HALF), f32),
            pltpu.VMEM((NROW, HALF), f32),
            pltpu.VMEM((NPT,), f32),
            pltpu.SemaphoreType.DMA,
            pltpu.SemaphoreType.DMA,
            pltpu.SemaphoreType.DMA,
            pltpu.SemaphoreType.DMA,
            pltpu.SemaphoreType.DMA,
            pltpu.SemaphoreType.DMA,
            pltpu.SemaphoreType.DMA,
        ],
    )
    out_halves, _ = run(xm, ccov, bcov, gcol5, row5)
    out = out_halves.transpose(1, 0, 2).reshape(NPAD, D_FEAT)[:N_NODES]
    return jnp.where(mask[:, None], x, out)
